# Initial kernel scaffold; baseline (speedup 1.0000x reference)
import jax
import jax.numpy as jnp
from jax.experimental import pallas as pl


def kernel(x, adj, C, W1, Wp, Wc, Wb):
    raise NotImplementedError("write your pallas kernel here")


# speedup vs baseline: 1.0562x; 1.0562x over previous
import jax
import jax.numpy as jnp
from jax import lax
from jax.experimental import pallas as pl
from jax.experimental.pallas import tpu as pltpu

N = 4096
BM = 256
NB = N // BM


def _cluster_kernel(x_ref, c_ref, wp_ref, wb_ref, hb_ref):
    x = x_ref[...]
    c = c_ref[...]
    cx = lax.dot_general(c, x, (((0,), (0,)), ((), ())),
                         preferred_element_type=jnp.float32)
    h1 = jnp.maximum(jnp.dot(cx, wp_ref[...],
                             preferred_element_type=jnp.float32), 0.0)
    g = jnp.dot(h1, wb_ref[...], preferred_element_type=jnp.float32)
    colsum = jnp.sum(c, axis=0)[:, None]
    hb_ref[...] = jnp.dot(c, g / colsum, preferred_element_type=jnp.float32)


def _fused_kernel(adj_ref, x_ref, hb_ref, w1_ref, wc_ref,
                  mu_ref, adj_bf, x1_bf, x2_s):
    t = pl.program_id(0)

    @pl.when(t < NB)
    def _phase0():
        i = t
        a = adj_ref[...]
        adj_bf[pl.ds(i * BM, BM), :] = a.astype(jnp.bfloat16)
        y = jnp.dot(a, x_ref[...], preferred_element_type=jnp.float32)
        x1 = jnp.maximum(
            jnp.dot(y, w1_ref[...], preferred_element_type=jnp.float32), 0.0)
        x1_bf[pl.ds(i * BM, BM), :] = x1.astype(jnp.bfloat16)

    @pl.when((t >= NB) & (t < 2 * NB))
    def _phase1():
        i = t - NB
        a_bf = adj_bf[pl.ds(i * BM, BM), :]
        y = jnp.dot(a_bf, x1_bf[...], preferred_element_type=jnp.float32)
        x2_s[pl.ds(i * BM, BM), :] = jnp.maximum(
            jnp.dot(y, wc_ref[...], preferred_element_type=jnp.float32)
            + hb_ref[pl.ds(i * BM, BM), :], 0.0)

    @pl.when(t >= 2 * NB)
    def _phase2():
        i = t - 2 * NB
        zb = x2_s[pl.ds(i * BM, BM), :]
        mu_ref[...] = jnp.maximum(
            lax.dot_general(zb, x2_s[...], (((1,), (1,)), ((), ())),
                            preferred_element_type=jnp.float32), 0.0)


def kernel(x, adj, C, W1, Wp, Wc, Wb):
    B, n, D = x.shape
    K = C.shape[1]
    x2d = x[0]

    hb = pl.pallas_call(
        _cluster_kernel,
        out_shape=jax.ShapeDtypeStruct((N, D), jnp.float32),
    )(x2d, C, Wp, Wb)

    mu = pl.pallas_call(
        _fused_kernel,
        grid=(3 * NB,),
        in_specs=[
            pl.BlockSpec((BM, N), lambda t: (jnp.minimum(t, NB - 1), 0)),
            pl.BlockSpec((N, D), lambda t: (0, 0)),
            pl.BlockSpec((N, D), lambda t: (0, 0)),
            pl.BlockSpec((D, D), lambda t: (0, 0)),
            pl.BlockSpec((D, D), lambda t: (0, 0)),
        ],
        out_specs=pl.BlockSpec((BM, N),
                               lambda t: (jnp.maximum(t - 2 * NB, 0), 0)),
        out_shape=jax.ShapeDtypeStruct((N, N), jnp.float32),
        scratch_shapes=[
            pltpu.VMEM((N, N), jnp.bfloat16),
            pltpu.VMEM((N, D), jnp.bfloat16),
            pltpu.VMEM((N, D), jnp.float32),
        ],
        compiler_params=pltpu.CompilerParams(
            vmem_limit_bytes=100 * 1024 * 1024),
    )(adj, x2d, hb, W1, Wc)

    return (mu.reshape(B, N, N), x)


# speedup vs baseline: 1.1034x; 1.0447x over previous
import jax
import jax.numpy as jnp
from jax import lax
from jax.experimental import pallas as pl
from jax.experimental.pallas import tpu as pltpu

N = 4096
BM = 256
NB = N // BM
BD = 256
ND = N // BD
T_DEC = 2 * NB


def _fused_kernel(adj_ref, c_ref, x_ref, w1_ref, wp_ref, wc_ref, wb_ref,
                  mu_ref, adj_bf, c_bf, x1_bf, x2_s, hb_s, cx_s, colsum_s):
    t = pl.program_id(0)

    @pl.when(t < NB)
    def _phase0():
        i = t
        a = adj_ref[...]
        adj_bf[pl.ds(i * BM, BM), :] = a.astype(jnp.bfloat16)
        y = jnp.dot(a, x_ref[...], preferred_element_type=jnp.float32)
        x1 = jnp.maximum(
            jnp.dot(y, w1_ref[...], preferred_element_type=jnp.float32), 0.0)
        x1_bf[pl.ds(i * BM, BM), :] = x1.astype(jnp.bfloat16)

        c = c_ref[...]
        c_bf[pl.ds(i * BM, BM), :] = c.astype(jnp.bfloat16)
        xc = x_ref[pl.ds(i * BM, BM), :]
        cx = lax.dot_general(c, xc, (((0,), (0,)), ((), ())),
                             preferred_element_type=jnp.float32)
        ones = jnp.ones((BM, 1), jnp.float32)
        cs = lax.dot_general(c, ones, (((0,), (0,)), ((), ())),
                             preferred_element_type=jnp.float32)

        @pl.when(t == 0)
        def _init():
            cx_s[...] = cx
            colsum_s[...] = cs

        @pl.when(t > 0)
        def _acc():
            cx_s[...] += cx
            colsum_s[...] += cs

    @pl.when(t == NB)
    def _cluster_finish():
        h1 = jnp.maximum(jnp.dot(cx_s[...], wp_ref[...],
                                 preferred_element_type=jnp.float32), 0.0)
        g = jnp.dot(h1, wb_ref[...], preferred_element_type=jnp.float32)
        gs = (g / colsum_s[...]).astype(jnp.bfloat16)
        hb_s[...] = jnp.dot(c_bf[...], gs, preferred_element_type=jnp.float32)

    @pl.when((t >= NB) & (t < 2 * NB))
    def _phase1():
        i = t - NB
        a_bf = adj_bf[pl.ds(i * BM, BM), :]
        y = jnp.dot(a_bf, x1_bf[...], preferred_element_type=jnp.float32)
        x2_s[pl.ds(i * BM, BM), :] = jnp.maximum(
            jnp.dot(y, wc_ref[...], preferred_element_type=jnp.float32)
            + hb_s[pl.ds(i * BM, BM), :], 0.0)

    @pl.when(t >= 2 * NB)
    def _phase2():
        i = t - T_DEC
        zb = x2_s[pl.ds(i * BD, BD), :]
        mu_ref[...] = jnp.maximum(
            lax.dot_general(zb, x2_s[...], (((1,), (1,)), ((), ())),
                            preferred_element_type=jnp.float32), 0.0)


def kernel(x, adj, C, W1, Wp, Wc, Wb):
    B, n, D = x.shape
    K = C.shape[1]
    x2d = x[0]

    mu = pl.pallas_call(
        _fused_kernel,
        grid=(2 * NB + ND,),
        in_specs=[
            pl.BlockSpec((BM, N), lambda t: (jnp.minimum(t, NB - 1), 0)),
            pl.BlockSpec((BM, K), lambda t: (jnp.minimum(t, NB - 1), 0)),
            pl.BlockSpec((N, D), lambda t: (0, 0)),
            pl.BlockSpec((D, D), lambda t: (0, 0)),
            pl.BlockSpec((D, D), lambda t: (0, 0)),
            pl.BlockSpec((D, D), lambda t: (0, 0)),
            pl.BlockSpec((D, D), lambda t: (0, 0)),
        ],
        out_specs=pl.BlockSpec((BD, N),
                               lambda t: (jnp.maximum(t - T_DEC, 0), 0)),
        out_shape=jax.ShapeDtypeStruct((N, N), jnp.float32),
        scratch_shapes=[
            pltpu.VMEM((N, N), jnp.bfloat16),
            pltpu.VMEM((N, K), jnp.bfloat16),
            pltpu.VMEM((N, D), jnp.bfloat16),
            pltpu.VMEM((N, D), jnp.float32),
            pltpu.VMEM((N, D), jnp.float32),
            pltpu.VMEM((K, D), jnp.float32),
            pltpu.VMEM((K, 1), jnp.float32),
        ],
        compiler_params=pltpu.CompilerParams(
            vmem_limit_bytes=63 * 1024 * 1024),
    )(adj, C, x2d, W1, Wp, Wc, Wb)

    return (mu.reshape(B, N, N), x)
